# baseline (speedup 1.0000x reference)
import functools

import jax
import jax.numpy as jnp
from jax.experimental import pallas as pl

_NUM_HEADS = 16
_QB = 512


def _mm_body(x_ref, w_ref, b_ref, o_ref):
    xb = x_ref[...].astype(jnp.bfloat16)
    wb = w_ref[...].astype(jnp.bfloat16)
    acc = jax.lax.dot_general(xb, wb, (((1,), (1,)), ((), ())),
                              preferred_element_type=jnp.float32)
    o_ref[...] = acc + b_ref[0:1, :]


def _matmul_bias(xm, w, b, bm, bn):
    M, K = xm.shape
    Nout = w.shape[0]
    bm = min(bm, M)
    bn = min(bn, Nout)
    bias = jnp.broadcast_to(b[None, :], (8, Nout))
    grid = (Nout // bn, M // bm)
    return pl.pallas_call(
        _mm_body,
        grid=grid,
        in_specs=[
            pl.BlockSpec((bm, K), lambda n, m: (m, 0)),
            pl.BlockSpec((bn, K), lambda n, m: (n, 0)),
            pl.BlockSpec((8, bn), lambda n, m: (0, n)),
        ],
        out_specs=pl.BlockSpec((bm, bn), lambda n, m: (m, n)),
        out_shape=jax.ShapeDtypeStruct((M, Nout), jnp.float32),
    )(xm, w, bias)


def _attn_body(q_ref, k_ref, v_ref, o_ref, *, kf, scale, max_iters):
    qb = q_ref[0, 0].astype(jnp.bfloat16)
    kb = k_ref[0, 0].astype(jnp.bfloat16)
    s = jax.lax.dot_general(qb, kb, (((1,), (1,)), ((), ())),
                            preferred_element_type=jnp.float32) * scale

    rowmax = jnp.max(s, axis=1, keepdims=True)
    rowmin = jnp.min(s, axis=1, keepdims=True)
    n_total = jnp.full_like(rowmax, s.shape[1])

    def count_ge(t):
        return jnp.sum((s >= t).astype(jnp.float32), axis=1, keepdims=True)

    def probe(state, mid):
        lo, cl, hi, ch = state
        cm = count_ge(mid)
        take = cm >= kf
        lo2 = jnp.where(take, mid, lo)
        cl2 = jnp.where(take, cm, cl)
        hi2 = jnp.where(take, hi, mid)
        ch2 = jnp.where(take, ch, cm)
        return (lo2, cl2, hi2, ch2)

    def falsi_mid(state):
        lo, cl, hi, ch = state
        frac = (cl - kf) / jnp.maximum(cl - ch, 1.0)
        frac = jnp.clip(frac, 0.03, 0.97)
        return lo + frac * (hi - lo)

    def cond(c):
        it, state, done = c
        return jnp.logical_and(it < max_iters, jnp.logical_not(done))

    def body(c):
        it, state, done = c
        state = probe(state, falsi_mid(state))
        done2 = jnp.max(jnp.abs(state[1] - kf)) == 0.0
        return (it + 1, state, done2)

    init = (rowmin, n_total, rowmax, jnp.ones_like(rowmax))
    _, (lo, _, _, _), _ = jax.lax.while_loop(
        cond, body, (jnp.int32(0), init, jnp.array(False)))

    e = jnp.where(s >= lo, jnp.exp(s - rowmax), 0.0)
    denom = jnp.sum(e, axis=1, keepdims=True)
    p = (e / denom).astype(jnp.bfloat16)
    v = v_ref[0, 0].astype(jnp.bfloat16)
    o_ref[0, 0] = jax.lax.dot_general(p, v, (((1,), (0,)), ((), ())),
                                      preferred_element_type=jnp.float32)


def _attention(qkvh, B, N, C, H, hd, k_keep):
    nq = N // min(_QB, N)
    grid = (B, H, nq)
    qb = min(_QB, N)
    body = functools.partial(_attn_body, kf=float(k_keep),
                             scale=hd ** -0.5, max_iters=40)
    return pl.pallas_call(
        body,
        grid=grid,
        in_specs=[
            pl.BlockSpec((1, 1, qb, hd), lambda b, h, qi: (b, h, qi, 0)),
            pl.BlockSpec((1, 1, N, hd), lambda b, h, qi: (b, H + h, 0, 0)),
            pl.BlockSpec((1, 1, N, hd), lambda b, h, qi: (b, 2 * H + h, 0, 0)),
        ],
        out_specs=pl.BlockSpec((1, 1, qb, hd), lambda b, h, qi: (b, h, qi, 0)),
        out_shape=jax.ShapeDtypeStruct((B, H, N, hd), jnp.float32),
    )(qkvh, qkvh, qkvh)


def kernel(x, Wqkv, bqkv, Wproj, bproj, temperature):
    B, N, C = x.shape
    H = _NUM_HEADS
    hd = C // H
    k_keep = max(1, int(N * 0.5))

    qkv = _matmul_bias(x.reshape(B * N, C), Wqkv, bqkv, bm=512, bn=1536)
    qkvh = qkv.reshape(B, N, 3 * H, hd).transpose(0, 2, 1, 3)

    temp = jnp.clip(temperature, 0.01, None)
    qkvh = jnp.concatenate([qkvh[:, :H] / temp, qkvh[:, H:]], axis=1)

    attn = _attention(qkvh, B, N, C, H, hd, k_keep)
    y = attn.transpose(0, 2, 1, 3).reshape(B * N, C)

    out = _matmul_bias(y, Wproj, bproj, bm=1024, bn=1024)
    return out.reshape(B, N, C)


# speedup vs baseline: 1.1010x; 1.1010x over previous
import functools

import jax
import jax.numpy as jnp
from jax.experimental import pallas as pl

_NUM_HEADS = 16
_QB = 512


def _mm_body(x_ref, w_ref, b_ref, o_ref):
    xb = x_ref[...].astype(jnp.bfloat16)
    wb = w_ref[...].astype(jnp.bfloat16)
    acc = jax.lax.dot_general(xb, wb, (((1,), (1,)), ((), ())),
                              preferred_element_type=jnp.float32)
    o_ref[...] = acc + b_ref[0:1, :]


def _matmul_bias(xm, w, b, bm, bn):
    M, K = xm.shape
    Nout = w.shape[0]
    bm = min(bm, M)
    bn = min(bn, Nout)
    bias = jnp.broadcast_to(b[None, :], (8, Nout))
    grid = (Nout // bn, M // bm)
    return pl.pallas_call(
        _mm_body,
        grid=grid,
        in_specs=[
            pl.BlockSpec((bm, K), lambda n, m: (m, 0)),
            pl.BlockSpec((bn, K), lambda n, m: (n, 0)),
            pl.BlockSpec((8, bn), lambda n, m: (0, n)),
        ],
        out_specs=pl.BlockSpec((bm, bn), lambda n, m: (m, n)),
        out_shape=jax.ShapeDtypeStruct((M, Nout), jnp.float32),
    )(xm, w, bias)


def _attn_body(q_ref, k_ref, v_ref, t_ref, o_ref, *, kf, scale, max_iters):
    qb = (q_ref[0, 0] / t_ref[0, 0]).astype(jnp.bfloat16)
    kb = k_ref[0, 0].astype(jnp.bfloat16)
    s = jax.lax.dot_general(qb, kb, (((1,), (1,)), ((), ())),
                            preferred_element_type=jnp.float32) * scale

    rowmax = jnp.max(s, axis=1, keepdims=True)
    rowmin = jnp.min(s, axis=1, keepdims=True)
    n_total = jnp.full_like(rowmax, s.shape[1])

    def count_ge(t):
        return jnp.sum((s >= t).astype(jnp.float32), axis=1, keepdims=True)

    def probe(state, mid):
        lo, cl, hi, ch = state
        cm = count_ge(mid)
        take = cm >= kf
        lo2 = jnp.where(take, mid, lo)
        cl2 = jnp.where(take, cm, cl)
        hi2 = jnp.where(take, hi, mid)
        ch2 = jnp.where(take, ch, cm)
        return (lo2, cl2, hi2, ch2)

    def falsi_mid(state):
        lo, cl, hi, ch = state
        frac = (cl - kf) / jnp.maximum(cl - ch, 1.0)
        frac = jnp.clip(frac, 0.03, 0.97)
        return lo + frac * (hi - lo)

    def cond(c):
        it, state, done = c
        return jnp.logical_and(it < max_iters, jnp.logical_not(done))

    def body(c):
        it, state, done = c
        state = probe(state, falsi_mid(state))
        done2 = jnp.max(jnp.abs(state[1] - kf)) == 0.0
        return (it + 1, state, done2)

    init = (rowmin, n_total, rowmax, jnp.ones_like(rowmax))
    _, (lo, _, _, _), _ = jax.lax.while_loop(
        cond, body, (jnp.int32(0), init, jnp.array(False)))

    e = jnp.where(s >= lo, jnp.exp(s - rowmax), 0.0)
    denom = jnp.sum(e, axis=1, keepdims=True)
    v = v_ref[0, 0].astype(jnp.bfloat16)
    o = jax.lax.dot_general(e.astype(jnp.bfloat16), v,
                            (((1,), (0,)), ((), ())),
                            preferred_element_type=jnp.float32)
    o_ref[0, 0] = o / denom


def _attention(qkvh, temp, B, N, C, H, hd, k_keep):
    nq = N // min(_QB, N)
    grid = (B, H, nq)
    qb = min(_QB, N)
    tarr = jnp.broadcast_to(temp.reshape(1, 1), (8, 128))
    body = functools.partial(_attn_body, kf=float(k_keep),
                             scale=hd ** -0.5, max_iters=40)
    return pl.pallas_call(
        body,
        grid=grid,
        in_specs=[
            pl.BlockSpec((1, 1, qb, hd), lambda b, h, qi: (b, h, qi, 0)),
            pl.BlockSpec((1, 1, N, hd), lambda b, h, qi: (b, H + h, 0, 0)),
            pl.BlockSpec((1, 1, N, hd), lambda b, h, qi: (b, 2 * H + h, 0, 0)),
            pl.BlockSpec((8, 128), lambda b, h, qi: (0, 0)),
        ],
        out_specs=pl.BlockSpec((1, 1, qb, hd), lambda b, h, qi: (b, h, qi, 0)),
        out_shape=jax.ShapeDtypeStruct((B, H, N, hd), jnp.float32),
    )(qkvh, qkvh, qkvh, tarr)


def kernel(x, Wqkv, bqkv, Wproj, bproj, temperature):
    B, N, C = x.shape
    H = _NUM_HEADS
    hd = C // H
    k_keep = max(1, int(N * 0.5))

    qkv = _matmul_bias(x.reshape(B * N, C), Wqkv, bqkv, bm=512, bn=1536)
    qkvh = qkv.reshape(B, N, 3 * H, hd).transpose(0, 2, 1, 3)

    temp = jnp.clip(temperature, 0.01, None)
    attn = _attention(qkvh, temp, B, N, C, H, hd, k_keep)
    y = attn.transpose(0, 2, 1, 3).reshape(B * N, C)

    out = _matmul_bias(y, Wproj, bproj, bm=1024, bn=1024)
    return out.reshape(B, N, C)
